# baseline (speedup 1.0000x reference)
import jax
import jax.numpy as jnp
from jax import lax
from jax.experimental import pallas as pl
from jax.experimental.pallas import tpu as pltpu
from jax.experimental.pallas import tpu_sc as plsc

_B = 16
_T = 512
_D = 384
_L = 2048
_NC = 2
_NS = 16
_NW = _NC * _NS
_FRAMES_PER_TILE = _B * _L // _NW
_CHUNK = 128
_NCHUNK = _FRAMES_PER_TILE // _CHUNK
_NBUF = 2
_LANES = 16


def _lr_body(table_hbm, dur_hbm, ml_hbm, out_hbm, mel_hbm,
             cs_ref, dur_ref, idx_ref, ml_ref, mel_ref,
             bufs0, bufs1,
             gs0, gs1, ws0, ws1):
    bufs = (bufs0, bufs1)
    gsem = (gs0, gs1)
    wsem = (ws0, ws1)
    wid = lax.axis_index("s") * _NC + lax.axis_index("c")
    b = wid // 2
    half = (wid + b) % 2
    j0 = half * _FRAMES_PER_TILE
    out_base = b * _L + j0

    pltpu.sync_copy(dur_hbm.at[b], dur_ref)
    pltpu.sync_copy(ml_hbm, ml_ref)

    carry = jnp.int32(0)
    for i in range(_T // _LANES):
        v = dur_ref[pl.ds(i * _LANES, _LANES)]
        cs_ref[pl.ds(i * _LANES, _LANES)] = plsc.cumsum(v) + carry
        carry = carry + jnp.sum(v)
    mel_len = carry

    @pl.when(half == 0)
    def _():
        mel_ref[...] = jnp.full((_LANES,), mel_len, jnp.int32)
        pltpu.sync_copy(mel_ref, mel_hbm.at[b])

    ml_scalar = jnp.max(ml_ref[...])
    valid_local = jnp.clip(jnp.minimum(mel_len, ml_scalar) - j0, 0, _FRAMES_PER_TILE)
    valid_vec = jnp.full((_LANES,), valid_local, jnp.int32)
    lanes = lax.iota(jnp.int32, _LANES)
    row_off = b * _T

    def nvalid(k):
        return jnp.clip(valid_local - k * _CHUNK, 0, _CHUNK)

    def compute_chunk(k):
        for i in range(_CHUNK // _LANES):
            loc = k * _CHUNK + i * _LANES + lanes
            p = loc + j0
            lo = jnp.zeros((_LANES,), jnp.int32)
            hi = jnp.full((_LANES,), _T, jnp.int32)
            for _ in range(10):
                mid = (lo + hi) >> 1
                cm = plsc.load_gather(cs_ref, [mid])
                take = cm <= p
                lo = jnp.where(take, mid + 1, lo)
                hi = jnp.where(take, hi, mid)
            flat = jnp.where(loc < valid_vec, lo, p & (_T - 1)) + row_off
            idx_ref[k, pl.ds(i * _LANES, _LANES)] = flat

    lax.fori_loop(0, _NCHUNK, lambda k, c: (compute_chunk(k), c)[1], 0)

    def gather(k):
        @pl.when(nvalid(k) > 0)
        def _():
            pltpu.async_copy(
                table_hbm.at[idx_ref.at[k]], bufs[k % _NBUF], gsem[k % _NBUF])

    def gather_wait(k):
        @pl.when(nvalid(k) > 0)
        def _():
            pltpu.make_async_copy(
                table_hbm.at[idx_ref.at[k]], bufs[k % _NBUF],
                gsem[k % _NBUF]).wait()

    zeros16 = jnp.zeros((_LANES,), jnp.float32)

    def write(k):
        dst = out_hbm.at[pl.ds(out_base + k * _CHUNK, _CHUNK)]
        nv = nvalid(k)
        buf = bufs[k % _NBUF]

        @pl.when(nv < _CHUNK)
        def _():
            def zrow(r, carry):
                for c in range(_D // _LANES):
                    buf[r, pl.ds(c * _LANES, _LANES)] = zeros16
                return carry

            lax.fori_loop(nv, _CHUNK, zrow, 0)

        pltpu.async_copy(buf, dst, wsem[k % _NBUF])

    def write_wait(k):
        pltpu.make_async_copy(
            bufs[k % _NBUF],
            out_hbm.at[pl.ds(out_base + k * _CHUNK, _CHUNK)],
            wsem[k % _NBUF]).wait()

    for k in range(_NBUF - 1):
        gather(k)
    for k in range(_NCHUNK):
        n = k + _NBUF - 1
        if n < _NCHUNK:
            if k >= 1:
                write_wait(k - 1)
            gather(n)
        gather_wait(k)
        write(k)
    for k in range(_NCHUNK - _NBUF, _NCHUNK):
        write_wait(k)


def _sc_expand(table, durations, ml_vec):
    mesh = plsc.VectorSubcoreMesh(
        core_axis_name="c", subcore_axis_name="s",
        num_cores=_NC, num_subcores=_NS)
    fn = pl.kernel(
        _lr_body,
        out_type=(
            jax.ShapeDtypeStruct((_B * _L, _D), jnp.float32),
            jax.ShapeDtypeStruct((_B, _LANES), jnp.int32),
        ),
        mesh=mesh,
        compiler_params=pltpu.CompilerParams(needs_layout_passes=False),
        scratch_types=(
            [pltpu.VMEM((_T,), jnp.int32),
             pltpu.VMEM((_T,), jnp.int32),
             pltpu.VMEM((_NCHUNK, _CHUNK), jnp.int32),
             pltpu.VMEM((_LANES,), jnp.int32),
             pltpu.VMEM((_LANES,), jnp.int32)]
            + [pltpu.VMEM((_CHUNK, _D), jnp.float32)] * _NBUF
            + [pltpu.SemaphoreType.DMA] * (2 * _NBUF)
        ),
    )
    return fn(table, durations, ml_vec)


def kernel(encoder_out, durations, max_len):
    B, T, D = encoder_out.shape
    table = encoder_out.reshape(B * T, D)
    ml_vec = jnp.full((_LANES,), max_len, jnp.int32)
    out_flat, mel_mat = _sc_expand(table, durations, ml_vec)
    return out_flat.reshape(B, _L, D), mel_mat[:, 0]


# speedup vs baseline: 1.0312x; 1.0312x over previous
import jax
import jax.numpy as jnp
from jax import lax
from jax.experimental import pallas as pl
from jax.experimental.pallas import tpu as pltpu
from jax.experimental.pallas import tpu_sc as plsc

_B = 16
_T = 512
_D = 384
_L = 2048
_NC = 2
_NS = 16
_NW = _NC * _NS
_FRAMES_PER_TILE = _B * _L // _NW
_CHUNK = 64
_NCHUNK = _FRAMES_PER_TILE // _CHUNK
_NBUF = 5
_LANES = 16


def _lr_body(table_hbm, dur_hbm, ml_hbm, out_hbm, mel_hbm,
             cs_ref, dur_ref, idx_ref, ml_ref, mel_ref,
             bufs0, bufs1, bufs2, bufs3, bufs4,
             gs0, gs1, gs2, gs3, gs4, ws0, ws1, ws2, ws3, ws4):
    bufs = (bufs0, bufs1, bufs2, bufs3, bufs4)
    gsem = (gs0, gs1, gs2, gs3, gs4)
    wsem = (ws0, ws1, ws2, ws3, ws4)
    wid = lax.axis_index("s") * _NC + lax.axis_index("c")
    b = wid // 2
    half = (wid + b) % 2
    j0 = half * _FRAMES_PER_TILE
    out_base = b * _L + j0

    pltpu.sync_copy(dur_hbm.at[b], dur_ref)
    pltpu.sync_copy(ml_hbm, ml_ref)

    carry = jnp.int32(0)
    for i in range(_T // _LANES):
        v = dur_ref[pl.ds(i * _LANES, _LANES)]
        cs_ref[pl.ds(i * _LANES, _LANES)] = plsc.cumsum(v) + carry
        carry = carry + jnp.sum(v)
    mel_len = carry

    @pl.when(half == 0)
    def _():
        mel_ref[...] = jnp.full((_LANES,), mel_len, jnp.int32)
        pltpu.sync_copy(mel_ref, mel_hbm.at[b])

    ml_scalar = jnp.max(ml_ref[...])
    valid_local = jnp.clip(jnp.minimum(mel_len, ml_scalar) - j0, 0, _FRAMES_PER_TILE)
    valid_vec = jnp.full((_LANES,), valid_local, jnp.int32)
    lanes = lax.iota(jnp.int32, _LANES)
    row_off = b * _T

    def nvalid(k):
        return jnp.clip(valid_local - k * _CHUNK, 0, _CHUNK)

    def compute_chunk(k):
        for i in range(_CHUNK // _LANES):
            loc = k * _CHUNK + i * _LANES + lanes
            p = loc + j0
            lo = jnp.zeros((_LANES,), jnp.int32)
            hi = jnp.full((_LANES,), _T, jnp.int32)
            for _ in range(10):
                mid = (lo + hi) >> 1
                cm = plsc.load_gather(cs_ref, [mid])
                take = cm <= p
                lo = jnp.where(take, mid + 1, lo)
                hi = jnp.where(take, hi, mid)
            flat = jnp.where(loc < valid_vec, lo, p & (_T - 1)) + row_off
            idx_ref[k, pl.ds(i * _LANES, _LANES)] = flat

    lax.fori_loop(0, _NCHUNK, lambda k, c: (compute_chunk(k), c)[1], 0)

    def gather(k):
        @pl.when(nvalid(k) > 0)
        def _():
            pltpu.async_copy(
                table_hbm.at[idx_ref.at[k]], bufs[k % _NBUF], gsem[k % _NBUF])

    def gather_wait(k):
        @pl.when(nvalid(k) > 0)
        def _():
            pltpu.make_async_copy(
                table_hbm.at[idx_ref.at[k]], bufs[k % _NBUF],
                gsem[k % _NBUF]).wait()

    zeros16 = jnp.zeros((_LANES,), jnp.float32)

    def write(k):
        dst = out_hbm.at[pl.ds(out_base + k * _CHUNK, _CHUNK)]
        nv = nvalid(k)
        buf = bufs[k % _NBUF]

        @pl.when(nv < _CHUNK)
        def _():
            def zrow(r, carry):
                for c in range(_D // _LANES):
                    buf[r, pl.ds(c * _LANES, _LANES)] = zeros16
                return carry

            lax.fori_loop(nv, _CHUNK, zrow, 0)

        pltpu.async_copy(buf, dst, wsem[k % _NBUF])

    def write_wait(k):
        pltpu.make_async_copy(
            bufs[k % _NBUF],
            out_hbm.at[pl.ds(out_base + k * _CHUNK, _CHUNK)],
            wsem[k % _NBUF]).wait()

    for k in range(_NBUF - 1):
        gather(k)
    for k in range(_NCHUNK):
        n = k + _NBUF - 1
        if n < _NCHUNK:
            if k >= 1:
                write_wait(k - 1)
            gather(n)
        gather_wait(k)
        write(k)
    for k in range(_NCHUNK - _NBUF, _NCHUNK):
        write_wait(k)


def _sc_expand(table, durations, ml_vec):
    mesh = plsc.VectorSubcoreMesh(
        core_axis_name="c", subcore_axis_name="s",
        num_cores=_NC, num_subcores=_NS)
    fn = pl.kernel(
        _lr_body,
        out_type=(
            jax.ShapeDtypeStruct((_B * _L, _D), jnp.float32),
            jax.ShapeDtypeStruct((_B, _LANES), jnp.int32),
        ),
        mesh=mesh,
        compiler_params=pltpu.CompilerParams(needs_layout_passes=False),
        scratch_types=(
            [pltpu.VMEM((_T,), jnp.int32),
             pltpu.VMEM((_T,), jnp.int32),
             pltpu.VMEM((_NCHUNK, _CHUNK), jnp.int32),
             pltpu.VMEM((_LANES,), jnp.int32),
             pltpu.VMEM((_LANES,), jnp.int32)]
            + [pltpu.VMEM((_CHUNK, _D), jnp.float32)] * _NBUF
            + [pltpu.SemaphoreType.DMA] * (2 * _NBUF)
        ),
    )
    return fn(table, durations, ml_vec)


def kernel(encoder_out, durations, max_len):
    B, T, D = encoder_out.shape
    table = encoder_out.reshape(B * T, D)
    ml_vec = jnp.full((_LANES,), max_len, jnp.int32)
    out_flat, mel_mat = _sc_expand(table, durations, ml_vec)
    return out_flat.reshape(B, _L, D), mel_mat[:, 0]
